# baseline (speedup 1.0000x reference)
import functools

import jax
import jax.numpy as jnp
from jax import lax
from jax.experimental import pallas as pl
from jax.experimental.pallas import tpu as pltpu, tpu_sc as plsc

_N = 4096
_D = 512
_BLK = 256
_NC = 2
_NS = 16
_NW = _NC * _NS
_RPW = _N // _NW


def _tc_body(o1_ref, o2t_ref, keys_ref, pos_ref):
    r0 = pl.program_id(0) * _BLK
    o1 = o1_ref[...]
    o2t = o2t_ref[...]
    n1 = jnp.sum(o1 * o1, axis=1, keepdims=True)
    n2 = jnp.sum(o2t * o2t, axis=0, keepdims=True)
    dot = jnp.dot(o1, o2t, preferred_element_type=jnp.float32)
    d2 = n1 - 2.0 * dot + n2
    cols = jax.lax.broadcasted_iota(jnp.int32, (_BLK, _N), 1)
    rows = jax.lax.broadcasted_iota(jnp.int32, (_BLK, _N), 0) + r0
    diag = cols == rows
    d2 = jnp.where(diag, jnp.inf, d2)
    d2 = jnp.maximum(d2, 1e-12)
    keys_ref[...] = jax.lax.bitcast_convert_type(d2, jnp.int32)

    dmask = diag.astype(jnp.float32)
    dd = jnp.sum(dot * dmask, axis=1, keepdims=True)
    n2d = jnp.sum(n2 * dmask, axis=1, keepdims=True)
    pos_ref[...] = n1 + n2d - 2.0 * dd


def _splat(x):
    return jnp.full((16,), x, jnp.int32)


def _sc_body(keys_hbm, rn_hbm, out_hbm, kbuf, hist, hist2, rnv, chosen):
    wid = lax.axis_index("s") * _NC + lax.axis_index("c")
    base = wid * _RPW
    pltpu.sync_copy(rn_hbm.at[pl.ds(base, _RPW)], rnv)

    lanes = lax.iota(jnp.int32, 16)
    zeros16 = jnp.zeros((16,), jnp.int32)
    ones16 = jnp.full((16,), 1, jnp.int32)
    lane0 = lanes == 0

    def row_step(r, _):
        def z1(j, _):
            hist[pl.ds(j * 16, 16)] = zeros16
            return 0
        lax.fori_loop(0, 128, z1, 0)

        def z2(j, _):
            hist2[pl.ds(j * 16, 16)] = zeros16
            return 0
        lax.fori_loop(0, 64, z2, 0)

        pltpu.sync_copy(keys_hbm.at[base + r], kbuf)
        k_vec = plsc.load_gather(rnv, [_splat(r)])

        def p1(j, _):
            v = kbuf[pl.ds(j * 16, 16)]
            b = (v >> 20) & 2047
            p = ((b & 127) << 4) | (b >> 7)
            plsc.addupdate_scatter(hist, [p], ones16)
            return 0
        lax.fori_loop(0, 256, p1, 0)

        def vsum(j, acc):
            return acc + hist[pl.ds(j * 16, 16)]
        G = lax.fori_loop(0, 128, vsum, zeros16)
        C = plsc.cumsum(G)
        mk = C <= k_vec
        g = plsc.all_reduce_population_count(mk)
        below_g = _splat(jnp.max(jnp.where(mk, C, 0)))
        r1 = k_vec - below_g

        def wg1(t, carry):
            jcnt, below, run = carry
            idx = ((t * 16 + lanes) << 4) | g
            w = plsc.load_gather(hist, [idx])
            c = run + plsc.cumsum(w)
            m = c <= r1
            jcnt = jcnt + plsc.all_reduce_population_count(m)
            below = jnp.maximum(below, _splat(jnp.max(jnp.where(m, c, 0))))
            run = _splat(jnp.max(c))
            return jcnt, below, run
        jcnt, below1, _ = lax.fori_loop(0, 8, wg1, (zeros16, zeros16, zeros16))
        b1 = g * 128 + jcnt
        r2 = r1 - below1

        def p2(j, _):
            v = kbuf[pl.ds(j * 16, 16)]
            m = ((v >> 20) & 2047) == b1
            b = (v >> 10) & 1023
            p = ((b & 63) << 4) | (b >> 6)
            plsc.addupdate_scatter(hist2, [p], ones16, mask=m)
            return 0
        lax.fori_loop(0, 256, p2, 0)

        def vsum2(j, acc):
            return acc + hist2[pl.ds(j * 16, 16)]
        G2 = lax.fori_loop(0, 64, vsum2, zeros16)
        C2 = plsc.cumsum(G2)
        mk2 = C2 <= r2
        g2 = plsc.all_reduce_population_count(mk2)
        below_g2 = _splat(jnp.max(jnp.where(mk2, C2, 0)))
        r3 = r2 - below_g2

        def wg2(t, carry):
            jcnt2, run = carry
            idx = ((t * 16 + lanes) << 4) | g2
            w = plsc.load_gather(hist2, [idx])
            c = run + plsc.cumsum(w)
            m = c <= r3
            jcnt2 = jcnt2 + plsc.all_reduce_population_count(m)
            run = _splat(jnp.max(c))
            return jcnt2, run
        jcnt2, _ = lax.fori_loop(0, 4, wg2, (zeros16, zeros16))
        b2 = g2 * 64 + jcnt2

        bits = (b1 << 20) | (b2 << 10) | 512
        plsc.store_scatter(chosen, [_splat(r)], bits, mask=lane0)
        return 0

    lax.fori_loop(0, _RPW, row_step, 0)
    pltpu.sync_copy(chosen, out_hbm.at[pl.ds(base, _RPW)])


@jax.jit
def _run(output1, output2, rn):
    o2t = output2.T
    keys, pos = pl.pallas_call(
        _tc_body,
        grid=(_N // _BLK,),
        in_specs=[
            pl.BlockSpec((_BLK, _D), lambda i: (i, 0)),
            pl.BlockSpec((_D, _N), lambda i: (0, 0)),
        ],
        out_specs=[
            pl.BlockSpec((_BLK, _N), lambda i: (i, 0)),
            pl.BlockSpec((_BLK, 1), lambda i: (i, 0)),
        ],
        out_shape=[
            jax.ShapeDtypeStruct((_N, _N), jnp.int32),
            jax.ShapeDtypeStruct((_N, 1), jnp.float32),
        ],
    )(output1, o2t)

    sc_select = pl.kernel(
        _sc_body,
        out_type=jax.ShapeDtypeStruct((_N,), jnp.int32),
        mesh=plsc.VectorSubcoreMesh(core_axis_name="c", subcore_axis_name="s"),
        compiler_params=pltpu.CompilerParams(needs_layout_passes=False),
        scratch_types=[
            pltpu.VMEM((_N,), jnp.int32),
            pltpu.VMEM((2048,), jnp.int32),
            pltpu.VMEM((1024,), jnp.int32),
            pltpu.VMEM((_RPW,), jnp.int32),
            pltpu.VMEM((_RPW,), jnp.int32),
        ],
    )
    chosen_bits = sc_select(keys, rn)

    dist = jnp.sqrt(jax.lax.bitcast_convert_type(chosen_bits, jnp.float32))
    neg_loss = jnp.clip(2.0 - dist, 0.0, None)
    return jnp.mean(pos[:, 0]) + jnp.mean(neg_loss)


def kernel(output1, output2, quant):
    N = output1.shape[0]
    q = min(100, N - 1)
    rn = jax.random.randint(jax.random.key(1234), (N,), 0, q)
    rn = jnp.minimum(rn, quant - 1).astype(jnp.int32)
    return _run(output1, output2, rn)


# speedup vs baseline: 1.3848x; 1.3848x over previous
import functools

import jax
import jax.numpy as jnp
from jax import lax
from jax.experimental import pallas as pl
from jax.experimental.pallas import tpu as pltpu, tpu_sc as plsc

_N = 4096
_D = 512
_BLK = 256
_NC = 2
_NS = 16
_NW = _NC * _NS
_RPW = _N // _NW


def _tc_body(o1_ref, o2t_ref, keys_ref, pos_ref):
    r0 = pl.program_id(0) * _BLK
    o1 = o1_ref[...]
    o2t = o2t_ref[...]
    n1 = jnp.sum(o1 * o1, axis=1, keepdims=True)
    n2 = jnp.sum(o2t * o2t, axis=0, keepdims=True)
    dot = jnp.dot(o1, o2t, preferred_element_type=jnp.float32)
    d2 = n1 - 2.0 * dot + n2
    cols = jax.lax.broadcasted_iota(jnp.int32, (_BLK, _N), 1)
    rows = jax.lax.broadcasted_iota(jnp.int32, (_BLK, _N), 0) + r0
    diag = cols == rows
    d2 = jnp.where(diag, jnp.inf, d2)
    d2 = jnp.maximum(d2, 1e-12)
    keys_ref[...] = jax.lax.bitcast_convert_type(d2, jnp.int32)

    dmask = diag.astype(jnp.float32)
    dd = jnp.sum(dot * dmask, axis=1, keepdims=True)
    n2d = jnp.sum(n2 * dmask, axis=1, keepdims=True)
    pos_ref[...] = n1 + n2d - 2.0 * dd


def _splat(x):
    return jnp.full((16,), x, jnp.int32)


def _sc_body(keys_hbm, rn_hbm, out_hbm, kbuf, hist, hist2, rnv, chosen):
    wid = lax.axis_index("s") * _NC + lax.axis_index("c")
    base = wid * _RPW
    pltpu.sync_copy(rn_hbm.at[pl.ds(base, _RPW)], rnv)

    lanes = lax.iota(jnp.int32, 16)
    zeros16 = jnp.zeros((16,), jnp.int32)
    ones16 = jnp.full((16,), 1, jnp.int32)
    lane0 = lanes == 0

    def row_step(r, _):
        rb = r // 8
        rloc = r % 8

        @pl.when(rloc == 0)
        def _fetch():
            pltpu.sync_copy(keys_hbm.at[pl.ds(base + rb * 8, 8)], kbuf)

        def z1(j, _):
            hist[pl.ds(j * 16, 16)] = zeros16
            return 0
        lax.fori_loop(0, 128, z1, 0, unroll=16)

        def z2(j, _):
            hist2[pl.ds(j * 16, 16)] = zeros16
            return 0
        lax.fori_loop(0, 64, z2, 0, unroll=16)

        k_vec = plsc.load_gather(rnv, [_splat(r)])

        def p1(j, _):
            v = kbuf[rloc, pl.ds(j * 16, 16)]
            b = (v >> 20) & 2047
            p = ((b & 127) << 4) | (b >> 7)
            plsc.addupdate_scatter(hist, [p], ones16)
            return 0
        lax.fori_loop(0, 256, p1, 0, unroll=8)

        def vsum(j, acc):
            return acc + hist[pl.ds(j * 16, 16)]
        G = lax.fori_loop(0, 128, vsum, zeros16, unroll=16)
        C = plsc.cumsum(G)
        mk = C <= k_vec
        g = plsc.all_reduce_population_count(mk)
        below_g = _splat(jnp.max(jnp.where(mk, C, 0)))
        r1 = k_vec - below_g

        def wg1(t, carry):
            jcnt, below, run = carry
            idx = ((t * 16 + lanes) << 4) | g
            w = plsc.load_gather(hist, [idx])
            c = run + plsc.cumsum(w)
            m = c <= r1
            jcnt = jcnt + plsc.all_reduce_population_count(m)
            below = jnp.maximum(below, _splat(jnp.max(jnp.where(m, c, 0))))
            run = _splat(jnp.max(c))
            return jcnt, below, run
        jcnt, below1, _ = lax.fori_loop(0, 8, wg1, (zeros16, zeros16, zeros16),
                                        unroll=8)
        b1 = g * 128 + jcnt
        r2 = r1 - below1

        def p2(j, _):
            v = kbuf[rloc, pl.ds(j * 16, 16)]
            t = v >> 10
            m = (t >> 10) == b1
            p = ((t & 63) << 4) | ((t >> 6) & 15)
            plsc.addupdate_scatter(hist2, [p], ones16, mask=m)
            return 0
        lax.fori_loop(0, 256, p2, 0, unroll=8)

        def vsum2(j, acc):
            return acc + hist2[pl.ds(j * 16, 16)]
        G2 = lax.fori_loop(0, 64, vsum2, zeros16, unroll=16)
        C2 = plsc.cumsum(G2)
        mk2 = C2 <= r2
        g2 = plsc.all_reduce_population_count(mk2)
        below_g2 = _splat(jnp.max(jnp.where(mk2, C2, 0)))
        r3 = r2 - below_g2

        def wg2(t, carry):
            jcnt2, run = carry
            idx = ((t * 16 + lanes) << 4) | g2
            w = plsc.load_gather(hist2, [idx])
            c = run + plsc.cumsum(w)
            m = c <= r3
            jcnt2 = jcnt2 + plsc.all_reduce_population_count(m)
            run = _splat(jnp.max(c))
            return jcnt2, run
        jcnt2, _ = lax.fori_loop(0, 4, wg2, (zeros16, zeros16), unroll=4)
        b2 = g2 * 64 + jcnt2

        bits = (b1 << 20) | (b2 << 10) | 512
        plsc.store_scatter(chosen, [_splat(r)], bits, mask=lane0)
        return 0

    lax.fori_loop(0, _RPW, row_step, 0)
    pltpu.sync_copy(chosen, out_hbm.at[pl.ds(base, _RPW)])


@jax.jit
def _run(output1, output2, rn):
    o2t = output2.T
    keys, pos = pl.pallas_call(
        _tc_body,
        grid=(_N // _BLK,),
        in_specs=[
            pl.BlockSpec((_BLK, _D), lambda i: (i, 0)),
            pl.BlockSpec((_D, _N), lambda i: (0, 0)),
        ],
        out_specs=[
            pl.BlockSpec((_BLK, _N), lambda i: (i, 0)),
            pl.BlockSpec((_BLK, 1), lambda i: (i, 0)),
        ],
        out_shape=[
            jax.ShapeDtypeStruct((_N, _N), jnp.int32),
            jax.ShapeDtypeStruct((_N, 1), jnp.float32),
        ],
    )(output1, o2t)

    sc_select = pl.kernel(
        _sc_body,
        out_type=jax.ShapeDtypeStruct((_N,), jnp.int32),
        mesh=plsc.VectorSubcoreMesh(core_axis_name="c", subcore_axis_name="s"),
        compiler_params=pltpu.CompilerParams(needs_layout_passes=False),
        scratch_types=[
            pltpu.VMEM((8, _N), jnp.int32),
            pltpu.VMEM((2048,), jnp.int32),
            pltpu.VMEM((1024,), jnp.int32),
            pltpu.VMEM((_RPW,), jnp.int32),
            pltpu.VMEM((_RPW,), jnp.int32),
        ],
    )
    chosen_bits = sc_select(keys, rn)

    dist = jnp.sqrt(jax.lax.bitcast_convert_type(chosen_bits, jnp.float32))
    neg_loss = jnp.clip(2.0 - dist, 0.0, None)
    return jnp.mean(pos[:, 0]) + jnp.mean(neg_loss)


def kernel(output1, output2, quant):
    N = output1.shape[0]
    q = min(100, N - 1)
    rn = jax.random.randint(jax.random.key(1234), (N,), 0, q)
    rn = jnp.minimum(rn, quant - 1).astype(jnp.int32)
    return _run(output1, output2, rn)


# speedup vs baseline: 2.6638x; 1.9236x over previous
import functools

import jax
import jax.numpy as jnp
from jax import lax
from jax.experimental import pallas as pl
from jax.experimental.pallas import tpu as pltpu, tpu_sc as plsc

_N = 4096
_D = 512
_BLK = 256
_NC = 2
_NS = 16
_NW = _NC * _NS
_RPW = _N // _NW


def _tc_body(o1_ref, o2t_ref, keys_ref, pos_ref):
    r0 = pl.program_id(0) * _BLK
    o1 = o1_ref[...]
    o2t = o2t_ref[...]
    n1 = jnp.sum(o1 * o1, axis=1, keepdims=True)
    n2 = jnp.sum(o2t * o2t, axis=0, keepdims=True)
    dot = jnp.dot(o1, o2t, preferred_element_type=jnp.float32)
    d2 = n1 - 2.0 * dot + n2
    cols = jax.lax.broadcasted_iota(jnp.int32, (_BLK, _N), 1)
    rows = jax.lax.broadcasted_iota(jnp.int32, (_BLK, _N), 0) + r0
    diag = cols == rows
    d2 = jnp.where(diag, jnp.inf, d2)
    d2 = jnp.maximum(d2, 1e-12)
    keys_ref[...] = jax.lax.bitcast_convert_type(d2, jnp.int32)

    dmask = diag.astype(jnp.float32)
    dd = jnp.sum(dot * dmask, axis=1, keepdims=True)
    n2d = jnp.sum(n2 * dmask, axis=1, keepdims=True)
    pos_ref[...] = n1 + n2d - 2.0 * dd


def _splat(x):
    return jnp.full((16,), x, jnp.int32)


def _sc_body(keys_hbm, rn_hbm, out_hbm, kbuf, hist, hist2, rnv, chosen):
    wid = lax.axis_index("s") * _NC + lax.axis_index("c")
    base = wid * _RPW
    pltpu.sync_copy(rn_hbm.at[pl.ds(base, _RPW)], rnv)

    lanes = lax.iota(jnp.int32, 16)
    zeros16 = jnp.zeros((16,), jnp.int32)
    ones16 = jnp.full((16,), 1, jnp.int32)
    lane0 = lanes == 0

    def row_step(r, _):
        rb = r // 8
        rloc = r % 8

        @pl.when(rloc == 0)
        def _fetch():
            pltpu.sync_copy(keys_hbm.at[pl.ds(base + rb * 8, 8)], kbuf)

        @plsc.parallel_loop(0, 2048, step=16, unroll=8)
        def _z1(i):
            hist[pl.ds(i, 16)] = zeros16

        @plsc.parallel_loop(0, 1024, step=16, unroll=8)
        def _z2(i):
            hist2[pl.ds(i, 16)] = zeros16

        k_vec = plsc.load_gather(rnv, [_splat(r)])

        @plsc.parallel_loop(0, _N, step=16, unroll=8)
        def _p1(i):
            v = kbuf[rloc, pl.ds(i, 16)]
            b = (v >> 20) & 2047
            p = ((b & 127) << 4) | (b >> 7)
            plsc.addupdate_scatter(hist, [p], ones16)

        @plsc.parallel_loop(0, 2048, step=16, unroll=8, carry=zeros16)
        def G(i, acc):
            return acc + hist[pl.ds(i, 16)]
        C = plsc.cumsum(G)
        mk = C <= k_vec
        g = plsc.all_reduce_population_count(mk)
        below_g = _splat(jnp.max(jnp.where(mk, C, 0)))
        r1 = k_vec - below_g

        def wg1(t, carry):
            jcnt, below, run = carry
            idx = ((t * 16 + lanes) << 4) | g
            w = plsc.load_gather(hist, [idx])
            c = run + plsc.cumsum(w)
            m = c <= r1
            jcnt = jcnt + plsc.all_reduce_population_count(m)
            below = jnp.maximum(below, _splat(jnp.max(jnp.where(m, c, 0))))
            run = _splat(jnp.max(c))
            return jcnt, below, run
        jcnt, below1, _ = lax.fori_loop(0, 8, wg1, (zeros16, zeros16, zeros16),
                                        unroll=8)
        b1 = g * 128 + jcnt
        r2 = r1 - below1

        @plsc.parallel_loop(0, _N, step=16, unroll=8)
        def _p2(i):
            v = kbuf[rloc, pl.ds(i, 16)]
            t = v >> 10
            m = (t >> 10) == b1
            p = ((t & 63) << 4) | ((t >> 6) & 15)
            plsc.addupdate_scatter(hist2, [p], ones16, mask=m)

        @plsc.parallel_loop(0, 1024, step=16, unroll=8, carry=zeros16)
        def G2(i, acc):
            return acc + hist2[pl.ds(i, 16)]
        C2 = plsc.cumsum(G2)
        mk2 = C2 <= r2
        g2 = plsc.all_reduce_population_count(mk2)
        below_g2 = _splat(jnp.max(jnp.where(mk2, C2, 0)))
        r3 = r2 - below_g2

        def wg2(t, carry):
            jcnt2, run = carry
            idx = ((t * 16 + lanes) << 4) | g2
            w = plsc.load_gather(hist2, [idx])
            c = run + plsc.cumsum(w)
            m = c <= r3
            jcnt2 = jcnt2 + plsc.all_reduce_population_count(m)
            run = _splat(jnp.max(c))
            return jcnt2, run
        jcnt2, _ = lax.fori_loop(0, 4, wg2, (zeros16, zeros16), unroll=4)
        b2 = g2 * 64 + jcnt2

        bits = (b1 << 20) | (b2 << 10) | 512
        plsc.store_scatter(chosen, [_splat(r)], bits, mask=lane0)
        return 0

    lax.fori_loop(0, _RPW, row_step, 0)
    pltpu.sync_copy(chosen, out_hbm.at[pl.ds(base, _RPW)])


@jax.jit
def _run(output1, output2, rn):
    o2t = output2.T
    keys, pos = pl.pallas_call(
        _tc_body,
        grid=(_N // _BLK,),
        in_specs=[
            pl.BlockSpec((_BLK, _D), lambda i: (i, 0)),
            pl.BlockSpec((_D, _N), lambda i: (0, 0)),
        ],
        out_specs=[
            pl.BlockSpec((_BLK, _N), lambda i: (i, 0)),
            pl.BlockSpec((_BLK, 1), lambda i: (i, 0)),
        ],
        out_shape=[
            jax.ShapeDtypeStruct((_N, _N), jnp.int32),
            jax.ShapeDtypeStruct((_N, 1), jnp.float32),
        ],
    )(output1, o2t)

    sc_select = pl.kernel(
        _sc_body,
        out_type=jax.ShapeDtypeStruct((_N,), jnp.int32),
        mesh=plsc.VectorSubcoreMesh(core_axis_name="c", subcore_axis_name="s"),
        compiler_params=pltpu.CompilerParams(needs_layout_passes=False),
        scratch_types=[
            pltpu.VMEM((8, _N), jnp.int32),
            pltpu.VMEM((2048,), jnp.int32),
            pltpu.VMEM((1024,), jnp.int32),
            pltpu.VMEM((_RPW,), jnp.int32),
            pltpu.VMEM((_RPW,), jnp.int32),
        ],
    )
    chosen_bits = sc_select(keys, rn)

    dist = jnp.sqrt(jax.lax.bitcast_convert_type(chosen_bits, jnp.float32))
    neg_loss = jnp.clip(2.0 - dist, 0.0, None)
    return jnp.mean(pos[:, 0]) + jnp.mean(neg_loss)


def kernel(output1, output2, quant):
    N = output1.shape[0]
    q = min(100, N - 1)
    rn = jax.random.randint(jax.random.key(1234), (N,), 0, q)
    rn = jnp.minimum(rn, quant - 1).astype(jnp.int32)
    return _run(output1, output2, rn)


# speedup vs baseline: 2.9646x; 1.1129x over previous
import functools

import jax
import jax.numpy as jnp
from jax import lax
from jax.experimental import pallas as pl
from jax.experimental.pallas import tpu as pltpu, tpu_sc as plsc

_N = 4096
_D = 512
_BLK = 256
_NC = 2
_NS = 16
_NW = _NC * _NS
_RPW = _N // _NW


def _tc_body(o1_ref, o2t_ref, keys_ref, pos_ref):
    r0 = pl.program_id(0) * _BLK
    o1 = o1_ref[...]
    o2t = o2t_ref[...]
    n1 = jnp.sum(o1 * o1, axis=1, keepdims=True)
    n2 = jnp.sum(o2t * o2t, axis=0, keepdims=True)
    dot = jnp.dot(o1, o2t, preferred_element_type=jnp.float32)
    d2 = n1 - 2.0 * dot + n2
    cols = jax.lax.broadcasted_iota(jnp.int32, (_BLK, _N), 1)
    rows = jax.lax.broadcasted_iota(jnp.int32, (_BLK, _N), 0) + r0
    diag = cols == rows
    d2 = jnp.where(diag, jnp.inf, d2)
    d2 = jnp.maximum(d2, 1e-12)
    keys_ref[...] = jax.lax.bitcast_convert_type(d2, jnp.int32)

    dmask = diag.astype(jnp.float32)
    dd = jnp.sum(dot * dmask, axis=1, keepdims=True)
    n2d = jnp.sum(n2 * dmask, axis=1, keepdims=True)
    pos_ref[...] = n1 + n2d - 2.0 * dd


def _splat(x):
    return jnp.full((16,), x, jnp.int32)


_GDN = jax.lax.GatherDimensionNumbers(
    offset_dims=(), collapsed_slice_dims=(0,), start_index_map=(0,))


def _lane_gather(x, idx_vec):
    return jax.lax.gather(x, idx_vec[:, None], _GDN, (1,),
                          mode=jax.lax.GatherScatterMode.PROMISE_IN_BOUNDS)


def _last_lane(x):
    return _lane_gather(x, jnp.full((16,), 15, jnp.int32))


def _sc_body(keys_hbm, rn_hbm, out_hbm, kbuf, hist, hist2, rnv, chosen, sems):
    wid = lax.axis_index("s") * _NC + lax.axis_index("c")
    base = wid * _RPW
    pltpu.sync_copy(rn_hbm.at[pl.ds(base, _RPW)], rnv)

    lanes = lax.iota(jnp.int32, 16)
    zeros16 = jnp.zeros((16,), jnp.int32)
    ones16 = jnp.full((16,), 1, jnp.int32)
    lane0 = lanes == 0

    def _block_copy(rb, par):
        return pltpu.make_async_copy(
            keys_hbm.at[pl.ds(base + rb * 8, 8)], kbuf.at[par], sems.at[par])

    _block_copy(0, 0).start()

    def block_step(rb, _):
        par = rb % 2
        _block_copy(rb, par).wait()

        @pl.when(rb < _RPW // 8 - 1)
        def _start_next():
            _block_copy(rb + 1, 1 - par).start()

        def row_step(rloc, _):
            r = rb * 8 + rloc
            _select_row(r, rloc, par)
            return 0
        lax.fori_loop(0, 8, row_step, 0)
        return 0

    def _select_row(r, rloc, par):
        @plsc.parallel_loop(0, 2048, step=16, unroll=8)
        def _z1(i):
            hist[pl.ds(i, 16)] = zeros16

        @plsc.parallel_loop(0, 1024, step=16, unroll=8)
        def _z2(i):
            hist2[pl.ds(i, 16)] = zeros16

        k_vec = plsc.load_gather(rnv, [_splat(r)])

        @plsc.parallel_loop(0, _N, step=16, unroll=8)
        def _p1(i):
            v = kbuf[par, rloc, pl.ds(i, 16)]
            b = (v >> 20) & 2047
            p = ((b & 127) << 4) | (b >> 7)
            plsc.addupdate_scatter(hist, [p], ones16)

        @plsc.parallel_loop(0, 2048, step=16, unroll=8, carry=zeros16)
        def G(i, acc):
            return acc + hist[pl.ds(i, 16)]
        g, r1 = _find_group(G, k_vec)

        jcnt, below1 = _within_group(hist, g, r1, 8)
        b1 = g * 128 + jcnt
        r2 = r1 - below1

        @plsc.parallel_loop(0, _N, step=16, unroll=8)
        def _p2(i):
            v = kbuf[par, rloc, pl.ds(i, 16)]
            t = v >> 10
            m = (t >> 10) == b1
            p = ((t & 63) << 4) | ((t >> 6) & 15)
            plsc.addupdate_scatter(hist2, [p], ones16, mask=m)

        @plsc.parallel_loop(0, 1024, step=16, unroll=8, carry=zeros16)
        def G2(i, acc):
            return acc + hist2[pl.ds(i, 16)]
        g2, r3 = _find_group(G2, r2)
        jcnt2, _ = _within_group(hist2, g2, r3, 4)
        b2 = g2 * 64 + jcnt2

        bits = (b1 << 20) | (b2 << 10) | 512
        plsc.store_scatter(chosen, [_splat(r)], bits, mask=lane0)

    def _find_group(G, rank):
        C = plsc.cumsum(G)
        mk = C <= rank
        g = plsc.all_reduce_population_count(mk)
        below = jnp.where(g == 0, 0,
                          _lane_gather(C, jnp.maximum(g - 1, 0)))
        return g, rank - below

    def _within_group(h, g, rank, nt):
        cs = [plsc.cumsum(plsc.load_gather(h, [((t * 16 + lanes) << 4) | g]))
              for t in range(nt)]
        run = zeros16
        jcnt = zeros16
        below_acc = zeros16
        for t in range(nt):
            c = run + cs[t]
            m = c <= rank
            jcnt = jcnt + plsc.all_reduce_population_count(m)
            below_acc = jnp.maximum(below_acc, jnp.where(m, c, 0))
            if t < nt - 1:
                run = _last_lane(c)
        below = _splat(jnp.max(below_acc))
        return jcnt, below

    lax.fori_loop(0, _RPW // 8, block_step, 0)
    pltpu.sync_copy(chosen, out_hbm.at[pl.ds(base, _RPW)])


@jax.jit
def _run(output1, output2, rn):
    o2t = output2.T
    keys, pos = pl.pallas_call(
        _tc_body,
        grid=(_N // _BLK,),
        in_specs=[
            pl.BlockSpec((_BLK, _D), lambda i: (i, 0)),
            pl.BlockSpec((_D, _N), lambda i: (0, 0)),
        ],
        out_specs=[
            pl.BlockSpec((_BLK, _N), lambda i: (i, 0)),
            pl.BlockSpec((_BLK, 1), lambda i: (i, 0)),
        ],
        out_shape=[
            jax.ShapeDtypeStruct((_N, _N), jnp.int32),
            jax.ShapeDtypeStruct((_N, 1), jnp.float32),
        ],
    )(output1, o2t)

    sc_select = pl.kernel(
        _sc_body,
        out_type=jax.ShapeDtypeStruct((_N,), jnp.int32),
        mesh=plsc.VectorSubcoreMesh(core_axis_name="c", subcore_axis_name="s"),
        compiler_params=pltpu.CompilerParams(needs_layout_passes=False),
        scratch_types=[
            pltpu.VMEM((2, 8, _N), jnp.int32),
            pltpu.VMEM((2048,), jnp.int32),
            pltpu.VMEM((1024,), jnp.int32),
            pltpu.VMEM((_RPW,), jnp.int32),
            pltpu.VMEM((_RPW,), jnp.int32),
            pltpu.SemaphoreType.DMA((2,)),
        ],
    )
    chosen_bits = sc_select(keys, rn)

    dist = jnp.sqrt(jax.lax.bitcast_convert_type(chosen_bits, jnp.float32))
    neg_loss = jnp.clip(2.0 - dist, 0.0, None)
    return jnp.mean(pos[:, 0]) + jnp.mean(neg_loss)


def kernel(output1, output2, quant):
    N = output1.shape[0]
    q = min(100, N - 1)
    rn = jax.random.randint(jax.random.key(1234), (N,), 0, q)
    rn = jnp.minimum(rn, quant - 1).astype(jnp.int32)
    return _run(output1, output2, rn)


# speedup vs baseline: 2.9696x; 1.0017x over previous
import functools

import jax
import jax.numpy as jnp
from jax import lax
from jax.experimental import pallas as pl
from jax.experimental.pallas import tpu as pltpu, tpu_sc as plsc

_N = 4096
_D = 512
_BLK = 256
_NC = 2
_NS = 16
_NW = _NC * _NS
_RPW = _N // _NW


def _tc_body(o1_ref, o2t_ref, keys_ref, pos_ref):
    r0 = pl.program_id(0) * _BLK
    o1 = o1_ref[...]
    o2t = o2t_ref[...]
    n1 = jnp.sum(o1 * o1, axis=1, keepdims=True)
    n2 = jnp.sum(o2t * o2t, axis=0, keepdims=True)
    dot = jnp.dot(o1, o2t, preferred_element_type=jnp.float32)
    d2 = n1 - 2.0 * dot + n2
    cols = jax.lax.broadcasted_iota(jnp.int32, (_BLK, _N), 1)
    rows = jax.lax.broadcasted_iota(jnp.int32, (_BLK, _N), 0) + r0
    diag = cols == rows
    d2 = jnp.where(diag, jnp.inf, d2)
    d2 = jnp.maximum(d2, 1e-12)
    keys_ref[...] = jax.lax.bitcast_convert_type(d2, jnp.int32)

    dmask = diag.astype(jnp.float32)
    dd = jnp.sum(dot * dmask, axis=1, keepdims=True)
    n2d = jnp.sum(n2 * dmask, axis=1, keepdims=True)
    pos_ref[...] = n1 + n2d - 2.0 * dd


def _splat(x):
    return jnp.full((16,), x, jnp.int32)


_GDN = jax.lax.GatherDimensionNumbers(
    offset_dims=(), collapsed_slice_dims=(0,), start_index_map=(0,))


def _lane_gather(x, idx_vec):
    return jax.lax.gather(x, idx_vec[:, None], _GDN, (1,),
                          mode=jax.lax.GatherScatterMode.PROMISE_IN_BOUNDS)


def _last_lane(x):
    return _lane_gather(x, jnp.full((16,), 15, jnp.int32))


def _sc_body(keys_hbm, rn_hbm, out_hbm, kbuf, hist, hist2, rnv, chosen, sems):
    wid = lax.axis_index("s") * _NC + lax.axis_index("c")
    base = wid * _RPW
    pltpu.sync_copy(rn_hbm.at[pl.ds(base, _RPW)], rnv)

    lanes = lax.iota(jnp.int32, 16)
    zeros16 = jnp.zeros((16,), jnp.int32)
    ones16 = jnp.full((16,), 1, jnp.int32)
    lane0 = lanes == 0

    def _block_copy(rb, par):
        return pltpu.make_async_copy(
            keys_hbm.at[pl.ds(base + rb * 8, 8)], kbuf.at[par], sems.at[par])

    _block_copy(0, 0).start()

    def block_step(rb, _):
        par = rb % 2
        _block_copy(rb, par).wait()

        @pl.when(rb < _RPW // 8 - 1)
        def _start_next():
            _block_copy(rb + 1, 1 - par).start()

        def row_step(rloc, _):
            r = rb * 8 + rloc
            _select_row(r, rloc, par)
            return 0
        lax.fori_loop(0, 8, row_step, 0)
        return 0

    def _select_row(r, rloc, par):
        @plsc.parallel_loop(0, 2048, step=16, unroll=8)
        def _z1(i):
            hist[pl.ds(i, 16)] = zeros16

        @plsc.parallel_loop(0, 1024, step=16, unroll=8)
        def _z2(i):
            hist2[pl.ds(i, 16)] = zeros16

        k_vec = plsc.load_gather(rnv, [_splat(r)])

        @plsc.parallel_loop(0, _N, step=16, unroll=8)
        def _p1(i):
            v = kbuf[par, rloc, pl.ds(i, 16)]
            b = (v >> 20) & 2047
            p = ((b & 127) << 4) | (b >> 7)
            cnt, last = plsc.scan_count(p)
            plsc.addupdate_scatter(hist, [p], cnt, mask=last)

        @plsc.parallel_loop(0, 2048, step=16, unroll=8, carry=zeros16)
        def G(i, acc):
            return acc + hist[pl.ds(i, 16)]
        g, r1 = _find_group(G, k_vec)

        jcnt, below1 = _within_group(hist, g, r1, 8)
        b1 = g * 128 + jcnt
        r2 = r1 - below1

        @plsc.parallel_loop(0, _N, step=16, unroll=8)
        def _p2(i):
            v = kbuf[par, rloc, pl.ds(i, 16)]
            t = v >> 10
            m = (t >> 10) == b1
            p = ((t & 63) << 4) | ((t >> 6) & 15)
            cnt, last = plsc.scan_count(p, mask=m)
            plsc.addupdate_scatter(hist2, [p], cnt, mask=last & m)

        @plsc.parallel_loop(0, 1024, step=16, unroll=8, carry=zeros16)
        def G2(i, acc):
            return acc + hist2[pl.ds(i, 16)]
        g2, r3 = _find_group(G2, r2)
        jcnt2, _ = _within_group(hist2, g2, r3, 4)
        b2 = g2 * 64 + jcnt2

        bits = (b1 << 20) | (b2 << 10) | 512
        plsc.store_scatter(chosen, [_splat(r)], bits, mask=lane0)

    def _find_group(G, rank):
        C = plsc.cumsum(G)
        mk = C <= rank
        g = plsc.all_reduce_population_count(mk)
        below = jnp.where(g == 0, 0,
                          _lane_gather(C, jnp.maximum(g - 1, 0)))
        return g, rank - below

    def _within_group(h, g, rank, nt):
        cs = [plsc.cumsum(plsc.load_gather(h, [((t * 16 + lanes) << 4) | g]))
              for t in range(nt)]
        run = zeros16
        jcnt = zeros16
        below_acc = zeros16
        for t in range(nt):
            c = run + cs[t]
            m = c <= rank
            jcnt = jcnt + plsc.all_reduce_population_count(m)
            below_acc = jnp.maximum(below_acc, jnp.where(m, c, 0))
            if t < nt - 1:
                run = _last_lane(c)
        below = _splat(jnp.max(below_acc))
        return jcnt, below

    lax.fori_loop(0, _RPW // 8, block_step, 0)
    pltpu.sync_copy(chosen, out_hbm.at[pl.ds(base, _RPW)])


@jax.jit
def _run(output1, output2, rn):
    o2t = output2.T
    keys, pos = pl.pallas_call(
        _tc_body,
        grid=(_N // _BLK,),
        in_specs=[
            pl.BlockSpec((_BLK, _D), lambda i: (i, 0)),
            pl.BlockSpec((_D, _N), lambda i: (0, 0)),
        ],
        out_specs=[
            pl.BlockSpec((_BLK, _N), lambda i: (i, 0)),
            pl.BlockSpec((_BLK, 1), lambda i: (i, 0)),
        ],
        out_shape=[
            jax.ShapeDtypeStruct((_N, _N), jnp.int32),
            jax.ShapeDtypeStruct((_N, 1), jnp.float32),
        ],
    )(output1, o2t)

    sc_select = pl.kernel(
        _sc_body,
        out_type=jax.ShapeDtypeStruct((_N,), jnp.int32),
        mesh=plsc.VectorSubcoreMesh(core_axis_name="c", subcore_axis_name="s"),
        compiler_params=pltpu.CompilerParams(needs_layout_passes=False),
        scratch_types=[
            pltpu.VMEM((2, 8, _N), jnp.int32),
            pltpu.VMEM((2048,), jnp.int32),
            pltpu.VMEM((1024,), jnp.int32),
            pltpu.VMEM((_RPW,), jnp.int32),
            pltpu.VMEM((_RPW,), jnp.int32),
            pltpu.SemaphoreType.DMA((2,)),
        ],
    )
    chosen_bits = sc_select(keys, rn)

    dist = jnp.sqrt(jax.lax.bitcast_convert_type(chosen_bits, jnp.float32))
    neg_loss = jnp.clip(2.0 - dist, 0.0, None)
    return jnp.mean(pos[:, 0]) + jnp.mean(neg_loss)


def kernel(output1, output2, quant):
    N = output1.shape[0]
    q = min(100, N - 1)
    rn = jax.random.randint(jax.random.key(1234), (N,), 0, q)
    rn = jnp.minimum(rn, quant - 1).astype(jnp.int32)
    return _run(output1, output2, rn)
